# baseline (speedup 1.0000x reference)
import functools

import jax
import jax.numpy as jnp
from jax import lax
from jax.experimental import pallas as pl
from jax.experimental.pallas import tpu as pltpu
from jax.experimental.pallas import tpu_sc as plsc

NUM_EMBEDDINGS = 8192
EMBED_DIM = 32
LANE_BLK = 1024

_NC, _NS = 2, 16
_NW = _NC * _NS
_GATHER_CHUNK = 128


def _vq_argmin_body(x_ref, emb_ref, idx_ref, esq_ref, emb2_ref):
    @pl.when(pl.program_id(0) == 0)
    def _():
        emb0 = emb_ref[...]
        esq_ref[...] = jnp.sum(emb0 * emb0, axis=1, keepdims=True)
        emb2_ref[...] = (emb0 * -2.0).astype(jnp.bfloat16)

    xb = x_ref[0].reshape(EMBED_DIM, LANE_BLK)
    xsq = jnp.sum(xb * xb, axis=0)
    s2 = lax.dot(
        emb2_ref[...],
        xb.astype(jnp.bfloat16),
        preferred_element_type=jnp.float32,
    )
    d2 = (xsq[None, :] + esq_ref[...]) + s2
    idx = jnp.argmin(d2, axis=0).astype(jnp.int32)
    idx_ref[0] = idx.reshape(LANE_BLK // _GATHER_CHUNK, _GATHER_CHUNK)


def _tc_argmin(x, embedding, grid):
    B, C, H, W = x.shape
    sub = LANE_BLK // _GATHER_CHUNK
    return pl.pallas_call(
        _vq_argmin_body,
        grid=(grid,),
        in_specs=[
            pl.BlockSpec((1, C, H, W), lambda i: (i, 0, 0, 0)),
            pl.BlockSpec((NUM_EMBEDDINGS, EMBED_DIM), lambda i: (0, 0)),
        ],
        out_specs=pl.BlockSpec((1, sub, _GATHER_CHUNK), lambda i: (i, 0, 0)),
        out_shape=jax.ShapeDtypeStruct((grid, sub, _GATHER_CHUNK), jnp.int32),
        scratch_shapes=[
            pltpu.VMEM((NUM_EMBEDDINGS, 1), jnp.float32),
            pltpu.VMEM((NUM_EMBEDDINGS, EMBED_DIM), jnp.bfloat16),
        ],
    )(x, embedding)


def _sc_gather(embedding, idx3d, n):
    rows_per_w = n // _NW
    n_chunks = rows_per_w // _GATHER_CHUNK
    mesh = plsc.VectorSubcoreMesh(core_axis_name="c", subcore_axis_name="s")

    @functools.partial(
        pl.kernel,
        mesh=mesh,
        out_type=jax.ShapeDtypeStruct((n, EMBED_DIM), jnp.float32),
        scratch_types=[
            pltpu.VMEM((n_chunks, _GATHER_CHUNK), jnp.int32),
            pltpu.VMEM((rows_per_w, EMBED_DIM), jnp.float32),
            pltpu.SemaphoreType.DMA,
        ],
        compiler_params=pltpu.CompilerParams(use_tc_tiling_on_sc=False),
    )
    def gather(table_hbm, idx_hbm, out_hbm, idx_v, rows_v, sem):
        wid = lax.axis_index("s") * _NC + lax.axis_index("c")
        pltpu.sync_copy(idx_hbm.at[wid], idx_v)
        copies = [
            pltpu.async_copy(
                table_hbm.at[idx_v.at[j]],
                rows_v.at[pl.ds(j * _GATHER_CHUNK, _GATHER_CHUNK)],
                sem,
            )
            for j in range(n_chunks)
        ]
        for c in copies:
            c.wait()
        pltpu.sync_copy(rows_v, out_hbm.at[pl.ds(wid * rows_per_w, rows_per_w)])

    return gather(embedding, idx3d)


def kernel(x, embedding):
    B, C, H, W = x.shape
    n = B * H * W
    grid = (B * H * W) // LANE_BLK

    idx3d = _tc_argmin(x, embedding, grid)
    embedding_b, idx3d = lax.optimization_barrier((embedding, idx3d))
    return _sc_gather(embedding_b, idx3d, n)
